# Initial kernel scaffold; baseline (speedup 1.0000x reference)
import jax
import jax.numpy as jnp
from jax.experimental import pallas as pl


def kernel(x, edge_index, edge_weight, W_ih_0, W_hh_0, b_ih_0, b_hh_0, W_ih_1, W_hh_1, b_ih_1, b_hh_1, b_gcn_0, b_gcn_1, fc1_W, fc1_b, fc2_W, fc2_b):
    raise NotImplementedError("write your pallas kernel here")


# speedup vs baseline: 1.2184x; 1.2184x over previous
import functools

import jax
import jax.numpy as jnp
from jax import lax
from jax.experimental import pallas as pl
from jax.experimental.pallas import tpu as pltpu

N = 50000
E = 800000
D = 32
WS = D * D
BLK = 5000
NSTEP = N // BLK


def _gru2_flat(W_ih, b_ih, b_hh):
    def cell(xv):
        gi = lax.dot_general(xv, W_ih, (((1,), (1,)), ((), ())))
        gi = gi + b_ih
        i_r, i_z, i_n = jnp.split(gi, 3, axis=1)
        h_r, h_z, h_n = jnp.split(b_hh, 3, axis=1)
        r = jax.nn.sigmoid(i_r + h_r)
        z = jax.nn.sigmoid(i_z + h_z)
        n = jnp.tanh(i_n + r * h_n)
        return (1.0 - z) * n
    h = cell(jnp.zeros((1, WS), jnp.float32))
    return cell(h)


def _gru_kernel(Wih0_ref, bih0_ref, bhh0_ref, Wih1_ref, bih1_ref, bhh1_ref,
                out_ref):
    h0 = _gru2_flat(Wih0_ref[...], bih0_ref[...], bhh0_ref[...])
    h1 = _gru2_flat(Wih1_ref[...], bih1_ref[...], bhh1_ref[...])
    out_ref[...] = jnp.concatenate([h0, h1], axis=0)


def _evolved_weights(W_ih_0, b_ih_0, b_hh_0, W_ih_1, b_ih_1, b_hh_1):
    h = pl.pallas_call(
        _gru_kernel,
        in_specs=[_full((3 * WS, WS)), _full((1, 3 * WS)), _full((1, 3 * WS)),
                  _full((3 * WS, WS)), _full((1, 3 * WS)), _full((1, 3 * WS))],
        out_specs=_full((2, WS)),
        out_shape=jax.ShapeDtypeStruct((2, WS), jnp.float32),
    )(W_ih_0, b_ih_0.reshape(1, 3 * WS), b_hh_0.reshape(1, 3 * WS),
      W_ih_1, b_ih_1.reshape(1, 3 * WS), b_hh_1.reshape(1, 3 * WS))
    return h[0].reshape(D, D), h[1].reshape(D, D)


def _fin_kernel(y2_ref, wf_ref, bg0_ref, W1_ref,
                bg1_ref, fc1W_ref, fc1b_ref, fc2W_ref, fc2b_ref, out_ref, r_acc):
    i = pl.program_id(0)

    @pl.when(i == 0)
    def _():
        r_acc[...] = jnp.zeros_like(r_acc)

    yblk = y2_ref[...]
    wrow = wf_ref[0, 0, :]
    b0 = bg0_ref[...]
    contribs = []
    for b in range(2):
        t = jax.nn.relu(yblk[b] + b0)
        contribs.append(lax.dot_general(wrow.reshape(1, BLK), t,
                                        (((1,), (0,)), ((), ()))))
    r_acc[...] = r_acc[...] + jnp.concatenate(contribs, axis=0)

    @pl.when(i == NSTEP - 1)
    def _():
        g = (r_acc[...] / float(N)) @ W1_ref[...] + bg1_ref[...]
        h1 = jax.nn.relu(g @ fc1W_ref[...] + fc1b_ref[...])
        out = jax.nn.sigmoid(h1 @ fc2W_ref[...] + fc2b_ref[...])
        out_ref[...] = out


def _full(shape):
    nd = len(shape)
    return pl.BlockSpec(shape, lambda *i: (0,) * nd)


def _finalize(y2, wf, b_gcn_0, W1, b_gcn_1, fc1_W, fc1_b, fc2_W, fc2_b):
    wf3 = wf.reshape(NSTEP, 1, BLK)
    return pl.pallas_call(
        _fin_kernel,
        grid=(NSTEP,),
        in_specs=[
            pl.BlockSpec((2, BLK, D), lambda i: (0, i, 0)),
            pl.BlockSpec((1, 1, BLK), lambda i: (i, 0, 0)),
            _full((1, D)),
            _full((D, D)),
            _full((1, D)),
            _full((D, 128)),
            _full((1, 128)),
            _full((128, 1)),
            _full((1, 1)),
        ],
        out_specs=_full((2, 1)),
        out_shape=jax.ShapeDtypeStruct((2, 1), jnp.float32),
        scratch_shapes=[pltpu.VMEM((2, D), jnp.float32)],
    )(y2, wf3, b_gcn_0.reshape(1, D), W1, b_gcn_1.reshape(1, D), fc1_W.T,
      fc1_b.reshape(1, 128), fc2_W.T, fc2_b.reshape(1, 1))


def kernel(x, edge_index, edge_weight, W_ih_0, W_hh_0, b_ih_0, b_hh_0,
           W_ih_1, W_hh_1, b_ih_1, b_hh_1, b_gcn_0, b_gcn_1,
           fc1_W, fc1_b, fc2_W, fc2_b):
    row, col = edge_index[0], edge_index[1]

    W0, W1 = _evolved_weights(W_ih_0, b_ih_0, b_hh_0, W_ih_1, b_ih_1, b_hh_1)

    deg = jnp.zeros((N,), jnp.float32).at[row].add(1.0)
    dis = jnp.where(deg > 0, lax.rsqrt(jnp.maximum(deg, 1.0)), 0.0)
    norm = dis[row] * edge_weight * dis[col]
    w = jnp.zeros((N,), jnp.float32).at[col].add(norm)

    x1 = x[:, 1].reshape(2 * N, D)
    sup = (x1 @ W0).reshape(2, N, D)
    msg = norm[:, None] * sup[:, col, :].transpose(1, 0, 2).reshape(E, 2 * D)
    y = jnp.zeros((N, 2 * D), jnp.float32).at[row].add(msg)
    y2 = y.reshape(N, 2, D).transpose(1, 0, 2)

    return _finalize(y2, w, b_gcn_0, W1, b_gcn_1, fc1_W, fc1_b, fc2_W, fc2_b)


# speedup vs baseline: 32.6620x; 26.8063x over previous
import functools

import jax
import jax.numpy as jnp
from jax import lax
from jax.experimental import pallas as pl
from jax.experimental.pallas import tpu as pltpu
from jax.experimental.pallas import tpu_sc as plsc

N = 50000
E = 800000
D = 32
WS = D * D

NC = 2
NS = 16
NW = NC * NS

CHUNK = 1024
KCH = 782
EPAD = KCH * CHUNK
NACC = 50048
RPT = NACC // NS

BLK = 3128
NSTEP = NACC // BLK


def _mesh():
    return plsc.VectorSubcoreMesh(core_axis_name="c", subcore_axis_name="s",
                                  num_cores=NC, num_subcores=NS)


_SC_PARAMS = pltpu.CompilerParams(needs_layout_passes=False,
                                  use_tc_tiling_on_sc=False)


def _wid():
    return lax.axis_index("s") * NC + lax.axis_index("c")


def _zero_fill(buf):
    n, = buf.shape
    zero = jnp.zeros((16,), jnp.float32)

    def body(i, carry):
        buf[pl.ds(i * 16, 16)] = zero
        return carry
    lax.fori_loop(0, n // 16, body, 0)


def _zero_fill2(buf):
    n, m = buf.shape
    zero = jnp.zeros((16,), jnp.float32)

    def body(i, carry):
        for q in range(m // 16):
            buf[i, pl.ds(q * 16, 16)] = zero
        return carry
    lax.fori_loop(0, n, body, 0)


def _sc_deg_body(row3, out, idx_v, ones_v, zbuf, acc, sem):
    c = lax.axis_index("c")
    s = lax.axis_index("s")
    w = _wid()

    _zero_fill(zbuf)
    one = jnp.ones((16,), jnp.float32)

    def ones_body(i, carry):
        ones_v[pl.ds(i * 16, 16)] = one
        return carry
    lax.fori_loop(0, 128 // 16, ones_body, 0)
    pltpu.sync_copy(zbuf.at[pl.ds(0, RPT)], acc.at[pl.ds(s * RPT, RPT)])
    plsc.subcore_barrier()

    def chunk_body(m, _):
        k = w + m * NW
        pltpu.sync_copy(row3.at[k], idx_v)
        descs = [pltpu.make_async_copy(ones_v, acc.at[idx_v.at[j]], sem)
                 for j in range(8)]
        for d in descs:
            d.start(add=True)
        for d in descs:
            d.wait()
        return 0

    nk = (KCH - w + NW - 1) // NW
    lax.fori_loop(0, nk, chunk_body, 0)

    plsc.subcore_barrier()
    pltpu.sync_copy(acc.at[pl.ds(s * RPT, RPT)], zbuf.at[pl.ds(0, RPT)])
    pltpu.sync_copy(zbuf.at[pl.ds(0, RPT)],
                    out.at[pl.ds(c * NACC + s * RPT, RPT)])


def _sc_deg(row3):
    f = pl.kernel(
        _sc_deg_body,
        out_type=jax.ShapeDtypeStruct((NC * NACC,), jnp.float32),
        mesh=_mesh(),
        compiler_params=_SC_PARAMS,
        scratch_types=[
            pltpu.VMEM((8, 128), jnp.int32),
            pltpu.VMEM((128,), jnp.float32),
            pltpu.VMEM((RPT,), jnp.float32),
            pltpu.VMEM_SHARED((NACC,), jnp.float32),
            pltpu.SemaphoreType.DMA,
        ],
    )
    return f(row3)


def _sc_norm_body(row3, col3, ew3, dis_hbm, norm3, wout,
                  dis_v, rbuf, cbuf, ebuf, nbuf, zbuf, acc, sem):
    c = lax.axis_index("c")
    s = lax.axis_index("s")
    w = _wid()

    pltpu.sync_copy(dis_hbm, dis_v)
    _zero_fill(zbuf)
    pltpu.sync_copy(zbuf.at[pl.ds(0, RPT)], acc.at[pl.ds(s * RPT, RPT)])
    plsc.subcore_barrier()

    def chunk_body(m, _):
        k = w + m * NW
        pltpu.sync_copy(row3.at[k], rbuf)
        pltpu.sync_copy(col3.at[k], cbuf)
        pltpu.sync_copy(ew3.at[k], ebuf)
        for j in range(8):
            for q in range(8):
                sl = pl.ds(q * 16, 16)
                r16 = rbuf[j, sl]
                c16 = cbuf[j, sl]
                dr = plsc.load_gather(dis_v, [r16])
                dc = plsc.load_gather(dis_v, [c16])
                nbuf[j, sl] = dr * ebuf[j, sl] * dc
        descs = [pltpu.make_async_copy(nbuf.at[j], acc.at[cbuf.at[j]], sem)
                 for j in range(8)]
        for d in descs:
            d.start(add=True)
        pltpu.sync_copy(nbuf, norm3.at[k])
        for d in descs:
            d.wait()
        return 0

    nk = (KCH - w + NW - 1) // NW
    lax.fori_loop(0, nk, chunk_body, 0)

    plsc.subcore_barrier()
    pltpu.sync_copy(acc.at[pl.ds(s * RPT, RPT)], zbuf.at[pl.ds(0, RPT)])
    pltpu.sync_copy(zbuf.at[pl.ds(0, RPT)],
                    wout.at[pl.ds(c * NACC + s * RPT, RPT)])


def _sc_norm(row3, col3, ew3, dis):
    f = pl.kernel(
        _sc_norm_body,
        out_type=[jax.ShapeDtypeStruct((KCH, 8, 128), jnp.float32),
                  jax.ShapeDtypeStruct((NC * NACC,), jnp.float32)],
        mesh=_mesh(),
        compiler_params=_SC_PARAMS,
        scratch_types=[
            pltpu.VMEM((NACC,), jnp.float32),
            pltpu.VMEM((8, 128), jnp.int32),
            pltpu.VMEM((8, 128), jnp.int32),
            pltpu.VMEM((8, 128), jnp.float32),
            pltpu.VMEM((8, 128), jnp.float32),
            pltpu.VMEM((RPT,), jnp.float32),
            pltpu.VMEM_SHARED((NACC,), jnp.float32),
            pltpu.SemaphoreType.DMA,
        ],
    )
    return f(row3, col3, ew3, dis)


def _sc_main_body(col3, norm3, row3, sup, yout,
                  cbuf, rbuf, nbuf, rows_v, mbuf, zbuf, acc, gsem, ssem):
    c = lax.axis_index("c")
    s = lax.axis_index("s")

    zrow = jnp.zeros((D,), jnp.bfloat16)

    def zfill(i, carry):
        zbuf[i] = zrow
        return carry
    lax.fori_loop(0, 184, zfill, 0)

    def zero_piece(p, carry):
        pltpu.sync_copy(zbuf, acc.at[pl.ds(s * RPT + p * 184, 184)])
        return carry
    lax.fori_loop(0, RPT // 184, zero_piece, 0)
    plsc.subcore_barrier()

    boff = (c * N).astype(jnp.int32)

    def chunk_body(m, _):
        k = s + m * NS
        pltpu.sync_copy(col3.at[k], cbuf)
        for j in range(8):
            for q in range(8):
                sl = pl.ds(q * 16, 16)
                cbuf[j, sl] = cbuf[j, sl] + boff
        gd = [pltpu.make_async_copy(sup.at[cbuf.at[j]],
                                    rows_v.at[pl.ds(j * 128, 128)], gsem)
              for j in range(8)]
        for d in gd:
            d.start()
        pltpu.sync_copy(norm3.at[k], nbuf)
        pltpu.sync_copy(row3.at[k], rbuf)
        for d in gd:
            d.wait()

        def scale_body(g, carry):
            j = g // 8
            q = g % 8
            nv = nbuf[j, pl.ds(q * 16, 16)]
            for u in range(16):
                r = g * 16 + u
                val = nv[u]
                v0 = rows_v[r, pl.ds(0, 16)] * val
                v1 = rows_v[r, pl.ds(16, 16)] * val
                mbuf[r] = plsc.pack(v0, v1, format=plsc.PackFormat.INTERLEAVED)
            return carry

        lax.fori_loop(0, CHUNK // 16, scale_body, 0)

        sd = [pltpu.make_async_copy(mbuf.at[pl.ds(j * 128, 128)],
                                    acc.at[rbuf.at[j]], ssem)
              for j in range(8)]
        for d in sd:
            d.start(add=True)
        for d in sd:
            d.wait()
        return 0

    nk = (KCH - s + NS - 1) // NS
    lax.fori_loop(0, nk, chunk_body, 0)

    plsc.subcore_barrier()

    def out_piece(p, carry):
        off = s * RPT + p * 184
        pltpu.sync_copy(acc.at[pl.ds(off, 184)], zbuf)
        pltpu.sync_copy(zbuf, yout.at[pl.ds(c * NACC + off, 184)])
        return carry
    lax.fori_loop(0, RPT // 184, out_piece, 0)


def _sc_main(col3, norm3, row3, sup):
    f = pl.kernel(
        _sc_main_body,
        out_type=jax.ShapeDtypeStruct((NC * NACC, D), jnp.bfloat16),
        mesh=_mesh(),
        compiler_params=_SC_PARAMS,
        scratch_types=[
            pltpu.VMEM((8, 128), jnp.int32),
            pltpu.VMEM((8, 128), jnp.int32),
            pltpu.VMEM((8, 128), jnp.float32),
            pltpu.VMEM((CHUNK, D), jnp.float32),
            pltpu.VMEM((CHUNK, D), jnp.bfloat16),
            pltpu.VMEM((184, D), jnp.bfloat16),
            pltpu.VMEM_SHARED((NACC, D), jnp.bfloat16),
            pltpu.SemaphoreType.DMA,
            pltpu.SemaphoreType.DMA,
        ],
    )
    return f(col3, norm3, row3, sup)


def _full(shape):
    nd = len(shape)
    return pl.BlockSpec(shape, lambda *i: (0,) * nd)


def _gru2_flat(W_ih, b_ih, b_hh):
    def cell(xv):
        gi = lax.dot_general(xv, W_ih, (((1,), (1,)), ((), ())))
        gi = gi + b_ih
        i_r, i_z, i_n = jnp.split(gi, 3, axis=1)
        h_r, h_z, h_n = jnp.split(b_hh, 3, axis=1)
        r = jax.nn.sigmoid(i_r + h_r)
        z = jax.nn.sigmoid(i_z + h_z)
        n = jnp.tanh(i_n + r * h_n)
        return (1.0 - z) * n
    h = cell(jnp.zeros((1, WS), jnp.float32))
    return cell(h)


def _gru_kernel(Wih0_ref, bih0_ref, bhh0_ref, Wih1_ref, bih1_ref, bhh1_ref,
                out_ref):
    h0 = _gru2_flat(Wih0_ref[...], bih0_ref[...], bhh0_ref[...])
    h1 = _gru2_flat(Wih1_ref[...], bih1_ref[...], bhh1_ref[...])
    out_ref[...] = jnp.concatenate([h0, h1], axis=0)


def _evolved_weights(W_ih_0, b_ih_0, b_hh_0, W_ih_1, b_ih_1, b_hh_1):
    h = pl.pallas_call(
        _gru_kernel,
        in_specs=[_full((3 * WS, WS)), _full((1, 3 * WS)), _full((1, 3 * WS)),
                  _full((3 * WS, WS)), _full((1, 3 * WS)), _full((1, 3 * WS))],
        out_specs=_full((2, WS)),
        out_shape=jax.ShapeDtypeStruct((2, WS), jnp.float32),
    )(W_ih_0, b_ih_0.reshape(1, 3 * WS), b_hh_0.reshape(1, 3 * WS),
      W_ih_1, b_ih_1.reshape(1, 3 * WS), b_hh_1.reshape(1, 3 * WS))
    return h[0].reshape(D, D), h[1].reshape(D, D)


SBLK = 10000


def _sup_kernel(x_ref, W_ref, out_ref):
    out_ref[...] = x_ref[...] @ W_ref[...]


def _support(x1cat, W0):
    return pl.pallas_call(
        _sup_kernel,
        grid=(2 * N // SBLK,),
        in_specs=[pl.BlockSpec((SBLK, D), lambda i: (i, 0)), _full((D, D))],
        out_specs=pl.BlockSpec((SBLK, D), lambda i: (i, 0)),
        out_shape=jax.ShapeDtypeStruct((2 * N, D), jnp.float32),
    )(x1cat, W0)


def _dis_kernel(degp_ref, out_ref):
    deg = degp_ref[0:1, :] + degp_ref[1:2, :]
    out_ref[...] = jnp.where(deg > 0, lax.rsqrt(jnp.maximum(deg, 1.0)), 0.0)


def _tc_dis(degp):
    return pl.pallas_call(
        _dis_kernel,
        in_specs=[_full((2, NACC))],
        out_specs=_full((1, NACC)),
        out_shape=jax.ShapeDtypeStruct((1, NACC), jnp.float32),
    )(degp)


def _fin_kernel(y2_ref, wf_ref, bg0_ref, W1_ref,
                bg1_ref, fc1W_ref, fc1b_ref, fc2W_ref, fc2b_ref, out_ref, r_acc):
    i = pl.program_id(0)

    @pl.when(i == 0)
    def _():
        r_acc[...] = jnp.zeros_like(r_acc)

    yblk = y2_ref[...].astype(jnp.float32)
    wrow = wf_ref[0, 0, 0, :] + wf_ref[1, 0, 0, :]
    b0 = bg0_ref[...]
    contribs = []
    for b in range(2):
        t = jax.nn.relu(yblk[b] + b0)
        contribs.append(lax.dot_general(wrow.reshape(1, BLK), t,
                                        (((1,), (0,)), ((), ()))))
    r_acc[...] = r_acc[...] + jnp.concatenate(contribs, axis=0)

    @pl.when(i == NSTEP - 1)
    def _():
        g = (r_acc[...] / float(N)) @ W1_ref[...] + bg1_ref[...]
        h1 = jax.nn.relu(g @ fc1W_ref[...] + fc1b_ref[...])
        out = jax.nn.sigmoid(h1 @ fc2W_ref[...] + fc2b_ref[...])
        out_ref[...] = out


def _finalize(y2, wp, b_gcn_0, W1, b_gcn_1, fc1_W, fc1_b, fc2_W, fc2_b):
    wf4 = wp.reshape(2, NSTEP, 1, BLK)
    return pl.pallas_call(
        _fin_kernel,
        grid=(NSTEP,),
        in_specs=[
            pl.BlockSpec((2, BLK, D), lambda i: (0, i, 0)),
            pl.BlockSpec((2, 1, 1, BLK), lambda i: (0, i, 0, 0)),
            _full((1, D)),
            _full((D, D)),
            _full((1, D)),
            _full((D, 128)),
            _full((1, 128)),
            _full((128, 1)),
            _full((1, 1)),
        ],
        out_specs=_full((2, 1)),
        out_shape=jax.ShapeDtypeStruct((2, 1), jnp.float32),
        scratch_shapes=[pltpu.VMEM((2, D), jnp.float32)],
    )(y2, wf4, b_gcn_0.reshape(1, D), W1, b_gcn_1.reshape(1, D), fc1_W.T,
      fc1_b.reshape(1, 128), fc2_W.T, fc2_b.reshape(1, 1))


def kernel(x, edge_index, edge_weight, W_ih_0, W_hh_0, b_ih_0, b_hh_0,
           W_ih_1, W_hh_1, b_ih_1, b_hh_1, b_gcn_0, b_gcn_1,
           fc1_W, fc1_b, fc2_W, fc2_b):
    row, col = edge_index[0], edge_index[1]

    npad = EPAD - E
    rowp = jnp.concatenate([row, N + (jnp.arange(npad, dtype=jnp.int32) % (NACC - N))])
    colp = jnp.concatenate([col, jnp.zeros((npad,), jnp.int32)])
    ewp = jnp.concatenate([edge_weight, jnp.zeros((npad,), jnp.float32)])
    row3 = rowp.reshape(KCH, 8, 128)
    col3 = colp.reshape(KCH, 8, 128)
    ew3 = ewp.reshape(KCH, 8, 128)

    W0, W1 = _evolved_weights(W_ih_0, b_ih_0, b_hh_0, W_ih_1, b_ih_1, b_hh_1)
    x1cat = x[:, 1].reshape(2 * N, D)
    sup = _support(x1cat, W0)

    degp = _sc_deg(row3).reshape(2, NACC)
    dis = _tc_dis(degp).reshape(NACC)
    norm3, wflat = _sc_norm(row3, col3, ew3, dis)
    wp = wflat.reshape(2, NACC)
    y2 = _sc_main(col3, norm3, row3, sup).reshape(2, NACC, D)

    perm = jnp.stack([jnp.arange(16, dtype=jnp.int32),
                      jnp.arange(16, dtype=jnp.int32) + 16], axis=1).reshape(32)
    b0p = b_gcn_0[perm]
    W1p = W1[perm, :]

    return _finalize(y2, wp, b0p, W1p, b_gcn_1, fc1_W, fc1_b, fc2_W, fc2_b)


# speedup vs baseline: 59.5179x; 1.8222x over previous
import functools

import jax
import jax.numpy as jnp
from jax import lax
from jax.experimental import pallas as pl
from jax.experimental.pallas import tpu as pltpu
from jax.experimental.pallas import tpu_sc as plsc

N = 50000
E = 800000
D = 32
WS = D * D

NC = 2
NS = 16
NW = NC * NS

CHUNK = 1024
KCH = 782
EPAD = KCH * CHUNK
NACC = 50048
RPT = NACC // NS

BLK = 3128
NSTEP = NACC // BLK


def _mesh():
    return plsc.VectorSubcoreMesh(core_axis_name="c", subcore_axis_name="s",
                                  num_cores=NC, num_subcores=NS)


_SC_PARAMS = pltpu.CompilerParams(needs_layout_passes=False,
                                  use_tc_tiling_on_sc=False)


def _wid():
    return lax.axis_index("s") * NC + lax.axis_index("c")


def _zero_fill(buf):
    n, = buf.shape
    zero = jnp.zeros((16,), jnp.float32)

    def body(i, carry):
        buf[pl.ds(i * 16, 16)] = zero
        return carry
    lax.fori_loop(0, n // 16, body, 0)


def _zero_fill2(buf):
    n, m = buf.shape
    zero = jnp.zeros((16,), jnp.float32)

    def body(i, carry):
        for q in range(m // 16):
            buf[i, pl.ds(q * 16, 16)] = zero
        return carry
    lax.fori_loop(0, n, body, 0)


def _sc_deg_body(row3, out, idx_v, ones_v, zbuf, acc, sem):
    c = lax.axis_index("c")
    s = lax.axis_index("s")
    w = _wid()

    _zero_fill(zbuf)
    one = jnp.ones((16,), jnp.float32)

    def ones_body(i, carry):
        ones_v[pl.ds(i * 16, 16)] = one
        return carry
    lax.fori_loop(0, 128 // 16, ones_body, 0)
    pltpu.sync_copy(zbuf.at[pl.ds(0, RPT)], acc.at[pl.ds(s * RPT, RPT)])
    plsc.subcore_barrier()

    def chunk_body(m, _):
        k = w + m * NW
        pltpu.sync_copy(row3.at[k], idx_v)
        descs = [pltpu.make_async_copy(ones_v, acc.at[idx_v.at[j]], sem)
                 for j in range(8)]
        for d in descs:
            d.start(add=True)
        for d in descs:
            d.wait()
        return 0

    nk = (KCH - w + NW - 1) // NW
    lax.fori_loop(0, nk, chunk_body, 0)

    plsc.subcore_barrier()
    pltpu.sync_copy(acc.at[pl.ds(s * RPT, RPT)], zbuf.at[pl.ds(0, RPT)])
    pltpu.sync_copy(zbuf.at[pl.ds(0, RPT)],
                    out.at[pl.ds(c * NACC + s * RPT, RPT)])


def _sc_deg(row3):
    f = pl.kernel(
        _sc_deg_body,
        out_type=jax.ShapeDtypeStruct((NC * NACC,), jnp.float32),
        mesh=_mesh(),
        compiler_params=_SC_PARAMS,
        scratch_types=[
            pltpu.VMEM((8, 128), jnp.int32),
            pltpu.VMEM((128,), jnp.float32),
            pltpu.VMEM((RPT,), jnp.float32),
            pltpu.VMEM_SHARED((NACC,), jnp.float32),
            pltpu.SemaphoreType.DMA,
        ],
    )
    return f(row3)


def _sc_norm_body(row3, col3, ew3, dis_hbm, norm3, wout,
                  dis_v, rbuf, cbuf, ebuf, nbuf, zbuf, acc, sem):
    c = lax.axis_index("c")
    s = lax.axis_index("s")
    w = _wid()

    pltpu.sync_copy(dis_hbm, dis_v)
    _zero_fill(zbuf)
    pltpu.sync_copy(zbuf.at[pl.ds(0, RPT)], acc.at[pl.ds(s * RPT, RPT)])
    plsc.subcore_barrier()

    def chunk_body(m, _):
        k = w + m * NW
        pltpu.sync_copy(row3.at[k], rbuf)
        pltpu.sync_copy(col3.at[k], cbuf)
        pltpu.sync_copy(ew3.at[k], ebuf)
        for j in range(8):
            for q in range(8):
                sl = pl.ds(q * 16, 16)
                r16 = rbuf[j, sl]
                c16 = cbuf[j, sl]
                dr = plsc.load_gather(dis_v, [r16])
                dc = plsc.load_gather(dis_v, [c16])
                nbuf[j, sl] = dr * ebuf[j, sl] * dc
        descs = [pltpu.make_async_copy(nbuf.at[j], acc.at[cbuf.at[j]], sem)
                 for j in range(8)]
        for d in descs:
            d.start(add=True)
        pltpu.sync_copy(nbuf, norm3.at[k])
        for d in descs:
            d.wait()
        return 0

    nk = (KCH - w + NW - 1) // NW
    lax.fori_loop(0, nk, chunk_body, 0)

    plsc.subcore_barrier()
    pltpu.sync_copy(acc.at[pl.ds(s * RPT, RPT)], zbuf.at[pl.ds(0, RPT)])
    pltpu.sync_copy(zbuf.at[pl.ds(0, RPT)],
                    wout.at[pl.ds(c * NACC + s * RPT, RPT)])


def _sc_norm(row3, col3, ew3, dis):
    f = pl.kernel(
        _sc_norm_body,
        out_type=[jax.ShapeDtypeStruct((KCH, 8, 128), jnp.float32),
                  jax.ShapeDtypeStruct((NC * NACC,), jnp.float32)],
        mesh=_mesh(),
        compiler_params=_SC_PARAMS,
        scratch_types=[
            pltpu.VMEM((NACC,), jnp.float32),
            pltpu.VMEM((8, 128), jnp.int32),
            pltpu.VMEM((8, 128), jnp.int32),
            pltpu.VMEM((8, 128), jnp.float32),
            pltpu.VMEM((8, 128), jnp.float32),
            pltpu.VMEM((RPT,), jnp.float32),
            pltpu.VMEM_SHARED((NACC,), jnp.float32),
            pltpu.SemaphoreType.DMA,
        ],
    )
    return f(row3, col3, ew3, dis)


def _sc_main_body(col3, norm3, row3, sup, yout,
                  cbuf, rbuf, nbuf, rows_v, zbuf, acc, lsem, gsem, ssem):
    c = lax.axis_index("c")
    s = lax.axis_index("s")

    zrow = jnp.zeros((D,), jnp.bfloat16)

    def zfill(i, carry):
        zbuf[i] = zrow
        return carry
    lax.fori_loop(0, 184, zfill, 0)

    def zero_piece(p, carry):
        pltpu.sync_copy(zbuf, acc.at[pl.ds(s * RPT + p * 184, 184)])
        return carry
    lax.fori_loop(0, RPT // 184, zero_piece, 0)
    plsc.subcore_barrier()

    boff = (c * N).astype(jnp.int32)
    nk = (KCH - s + NS - 1) // NS

    def kof(m):
        return s + jnp.minimum(m, nk - 1) * NS

    def lin_start(m):
        t = m % 3
        k = kof(m)
        ds = [pltpu.make_async_copy(col3.at[k], cbuf.at[t], lsem.at[t]),
              pltpu.make_async_copy(norm3.at[k], nbuf.at[t], lsem.at[t]),
              pltpu.make_async_copy(row3.at[k], rbuf.at[t], lsem.at[t])]
        for d in ds:
            d.start()

    def lin_wait(m):
        t = m % 3
        for _ in range(3):
            pltpu.make_async_copy(col3.at[0], cbuf.at[t], lsem.at[t]).wait()

    def gather_start(m):
        t = m % 3
        b = m % 2
        for j in range(8):
            for q in range(8):
                sl = pl.ds(q * 16, 16)
                cbuf[t, j, sl] = cbuf[t, j, sl] + boff
        for j in range(8):
            pltpu.make_async_copy(sup.at[cbuf.at[t, j]],
                                  rows_v.at[b, pl.ds(j * 128, 128)],
                                  gsem).start()

    def gather_wait():
        for j in range(8):
            pltpu.make_async_copy(sup.at[cbuf.at[0, 0]],
                                  rows_v.at[0, pl.ds(0, 128)], gsem).wait()

    def scale(m):
        t = m % 3
        b = m % 2

        def scale_body(g, carry):
            j = g // 8
            q = g % 8
            nv = nbuf[t, j, pl.ds(q * 16, 16)]
            for u in range(16):
                r = g * 16 + u
                val = nv[u]
                ra, rb = plsc.unpack(rows_v[b, r],
                                     format=plsc.PackFormat.INTERLEAVED)
                rows_v[b, r] = plsc.pack(ra * val, rb * val,
                                         format=plsc.PackFormat.INTERLEAVED)
            return carry

        lax.fori_loop(0, CHUNK // 16, scale_body, 0)

    def scatter_start(m):
        t = m % 3
        b = m % 2
        for j in range(8):
            pltpu.make_async_copy(rows_v.at[b, pl.ds(j * 128, 128)],
                                  acc.at[rbuf.at[t, j]], ssem).start(add=True)

    def scatter_wait():
        for j in range(8):
            pltpu.make_async_copy(rows_v.at[0, pl.ds(0, 128)],
                                  acc.at[rbuf.at[0, 0]], ssem).wait()

    lin_start(0)
    lin_start(1)
    lin_wait(0)
    gather_start(0)

    def chunk_iter(m, _):
        @pl.when(m > 0)
        def _():
            scatter_wait()
        lin_start(m + 2)
        gather_wait()
        lin_wait(m + 1)
        gather_start(m + 1)
        scale(m)
        scatter_start(m)
        return 0

    lax.fori_loop(0, nk, chunk_iter, 0)

    scatter_wait()
    gather_wait()
    lin_wait(nk + 1)

    plsc.subcore_barrier()

    def out_piece(p, carry):
        off = s * RPT + p * 184
        pltpu.sync_copy(acc.at[pl.ds(off, 184)], zbuf)
        pltpu.sync_copy(zbuf, yout.at[pl.ds(c * NACC + off, 184)])
        return carry
    lax.fori_loop(0, RPT // 184, out_piece, 0)


def _sc_main(col3, norm3, row3, sup):
    f = pl.kernel(
        _sc_main_body,
        out_type=jax.ShapeDtypeStruct((NC * NACC, D), jnp.bfloat16),
        mesh=_mesh(),
        compiler_params=_SC_PARAMS,
        scratch_types=[
            pltpu.VMEM((3, 8, 128), jnp.int32),
            pltpu.VMEM((3, 8, 128), jnp.int32),
            pltpu.VMEM((3, 8, 128), jnp.float32),
            pltpu.VMEM((2, CHUNK, D), jnp.bfloat16),
            pltpu.VMEM((184, D), jnp.bfloat16),
            pltpu.VMEM_SHARED((NACC, D), jnp.bfloat16),
            pltpu.SemaphoreType.DMA((3,)),
            pltpu.SemaphoreType.DMA,
            pltpu.SemaphoreType.DMA,
        ],
    )
    return f(col3, norm3, row3, sup)


def _full(shape):
    nd = len(shape)
    return pl.BlockSpec(shape, lambda *i: (0,) * nd)


def _gru2_flat(W_ih, b_ih, b_hh):
    def cell(xv):
        gi = lax.dot_general(xv, W_ih, (((1,), (1,)), ((), ())))
        gi = gi + b_ih
        i_r, i_z, i_n = jnp.split(gi, 3, axis=1)
        h_r, h_z, h_n = jnp.split(b_hh, 3, axis=1)
        r = jax.nn.sigmoid(i_r + h_r)
        z = jax.nn.sigmoid(i_z + h_z)
        n = jnp.tanh(i_n + r * h_n)
        return (1.0 - z) * n
    h = cell(jnp.zeros((1, WS), jnp.float32))
    return cell(h)


def _gru_kernel(Wih0_ref, bih0_ref, bhh0_ref, Wih1_ref, bih1_ref, bhh1_ref,
                out_ref):
    h0 = _gru2_flat(Wih0_ref[...], bih0_ref[...], bhh0_ref[...])
    h1 = _gru2_flat(Wih1_ref[...], bih1_ref[...], bhh1_ref[...])
    out_ref[...] = jnp.concatenate([h0, h1], axis=0)


def _evolved_weights(W_ih_0, b_ih_0, b_hh_0, W_ih_1, b_ih_1, b_hh_1):
    h = pl.pallas_call(
        _gru_kernel,
        in_specs=[_full((3 * WS, WS)), _full((1, 3 * WS)), _full((1, 3 * WS)),
                  _full((3 * WS, WS)), _full((1, 3 * WS)), _full((1, 3 * WS))],
        out_specs=_full((2, WS)),
        out_shape=jax.ShapeDtypeStruct((2, WS), jnp.float32),
    )(W_ih_0, b_ih_0.reshape(1, 3 * WS), b_hh_0.reshape(1, 3 * WS),
      W_ih_1, b_ih_1.reshape(1, 3 * WS), b_hh_1.reshape(1, 3 * WS))
    return h[0].reshape(D, D), h[1].reshape(D, D)


SBLK = 10000


def _sup_kernel(x_ref, W_ref, out_ref):
    out_ref[...] = (x_ref[...] @ W_ref[...]).astype(jnp.bfloat16)


def _support(x1cat, W0):
    return pl.pallas_call(
        _sup_kernel,
        grid=(2 * N // SBLK,),
        in_specs=[pl.BlockSpec((SBLK, D), lambda i: (i, 0)), _full((D, D))],
        out_specs=pl.BlockSpec((SBLK, D), lambda i: (i, 0)),
        out_shape=jax.ShapeDtypeStruct((2 * N, D), jnp.bfloat16),
    )(x1cat, W0)


def _dis_kernel(degp_ref, out_ref):
    deg = degp_ref[0:1, :] + degp_ref[1:2, :]
    out_ref[...] = jnp.where(deg > 0, lax.rsqrt(jnp.maximum(deg, 1.0)), 0.0)


def _tc_dis(degp):
    return pl.pallas_call(
        _dis_kernel,
        in_specs=[_full((2, NACC))],
        out_specs=_full((1, NACC)),
        out_shape=jax.ShapeDtypeStruct((1, NACC), jnp.float32),
    )(degp)


def _fin_kernel(y2_ref, wf_ref, bg0_ref, W1_ref,
                bg1_ref, fc1W_ref, fc1b_ref, fc2W_ref, fc2b_ref, out_ref, r_acc):
    i = pl.program_id(0)

    @pl.when(i == 0)
    def _():
        r_acc[...] = jnp.zeros_like(r_acc)

    yblk = y2_ref[...].astype(jnp.float32)
    wrow = wf_ref[0, 0, 0, :] + wf_ref[1, 0, 0, :]
    b0 = bg0_ref[...]
    contribs = []
    for b in range(2):
        t = jax.nn.relu(yblk[b] + b0)
        contribs.append(lax.dot_general(wrow.reshape(1, BLK), t,
                                        (((1,), (0,)), ((), ()))))
    r_acc[...] = r_acc[...] + jnp.concatenate(contribs, axis=0)

    @pl.when(i == NSTEP - 1)
    def _():
        g = (r_acc[...] / float(N)) @ W1_ref[...] + bg1_ref[...]
        h1 = jax.nn.relu(g @ fc1W_ref[...] + fc1b_ref[...])
        out = jax.nn.sigmoid(h1 @ fc2W_ref[...] + fc2b_ref[...])
        out_ref[...] = out


def _finalize(y2, wp, b_gcn_0, W1, b_gcn_1, fc1_W, fc1_b, fc2_W, fc2_b):
    wf4 = wp.reshape(2, NSTEP, 1, BLK)
    return pl.pallas_call(
        _fin_kernel,
        grid=(NSTEP,),
        in_specs=[
            pl.BlockSpec((2, BLK, D), lambda i: (0, i, 0)),
            pl.BlockSpec((2, 1, 1, BLK), lambda i: (0, i, 0, 0)),
            _full((1, D)),
            _full((D, D)),
            _full((1, D)),
            _full((D, 128)),
            _full((1, 128)),
            _full((128, 1)),
            _full((1, 1)),
        ],
        out_specs=_full((2, 1)),
        out_shape=jax.ShapeDtypeStruct((2, 1), jnp.float32),
        scratch_shapes=[pltpu.VMEM((2, D), jnp.float32)],
    )(y2, wf4, b_gcn_0.reshape(1, D), W1, b_gcn_1.reshape(1, D), fc1_W.T,
      fc1_b.reshape(1, 128), fc2_W.T, fc2_b.reshape(1, 1))


def kernel(x, edge_index, edge_weight, W_ih_0, W_hh_0, b_ih_0, b_hh_0,
           W_ih_1, W_hh_1, b_ih_1, b_hh_1, b_gcn_0, b_gcn_1,
           fc1_W, fc1_b, fc2_W, fc2_b):
    row, col = edge_index[0], edge_index[1]

    npad = EPAD - E
    rowp = jnp.concatenate([row, N + (jnp.arange(npad, dtype=jnp.int32) % (NACC - N))])
    colp = jnp.concatenate([col, jnp.zeros((npad,), jnp.int32)])
    ewp = jnp.concatenate([edge_weight, jnp.zeros((npad,), jnp.float32)])
    row3 = rowp.reshape(KCH, 8, 128)
    col3 = colp.reshape(KCH, 8, 128)
    ew3 = ewp.reshape(KCH, 8, 128)

    W0, W1 = _evolved_weights(W_ih_0, b_ih_0, b_hh_0, W_ih_1, b_ih_1, b_hh_1)
    x1cat = x[:, 1].reshape(2 * N, D)
    sup = _support(x1cat, W0)

    degp = _sc_deg(row3).reshape(2, NACC)
    dis = _tc_dis(degp).reshape(NACC)
    norm3, wflat = _sc_norm(row3, col3, ew3, dis)
    wp = wflat.reshape(2, NACC)
    y2 = _sc_main(col3, norm3, row3, sup).reshape(2, NACC, D)

    return _finalize(y2, wp, b_gcn_0, W1, b_gcn_1,
                     fc1_W, fc1_b, fc2_W, fc2_b)
